# Initial kernel scaffold; baseline (speedup 1.0000x reference)
import jax
import jax.numpy as jnp
from jax.experimental import pallas as pl


def kernel(E, q_weight, c_weight, q_bias, cls_bias):
    raise NotImplementedError("write your pallas kernel here")


# speedup vs baseline: 7.5045x; 7.5045x over previous
import jax
import jax.numpy as jnp
from jax import lax
from jax.experimental import pallas as pl

B, T, D = 2, 2048, 768
K = 16
NUM_CLASSES = 1000
G = 63
TOP_M = 8


def _body(e_ref, q_ref, c_ref, cb_ref, out_ref):
    E2 = e_ref[0]
    sq = jnp.where(q_ref[...] >= 0, 1.0, -1.0).astype(jnp.float32)
    scores = lax.dot_general(
        sq, E2, (((1,), (1,)), ((), ())),
        preferred_element_type=jnp.float32,
        precision=lax.Precision.HIGHEST,
    )

    iot = lax.broadcasted_iota(jnp.int32, (K, T), 1)

    def pick(i, carry):
        s, w = carry
        cur = jnp.max(s, axis=1, keepdims=True)
        eq = s == cur
        idx = jnp.min(jnp.where(eq, iot, T), axis=1, keepdims=True)
        sel = iot == idx
        w = w + sel.astype(jnp.float32)
        s = jnp.where(sel, -jnp.inf, s)
        return s, w

    _, W = lax.fori_loop(
        0, TOP_M, pick, (scores, jnp.zeros((K, T), jnp.float32))
    )

    g = lax.dot_general(
        W, E2, (((1,), (0,)), ((), ())),
        preferred_element_type=jnp.float32,
        precision=lax.Precision.HIGHEST,
    )

    signed = jnp.where(c_ref[...] >= 0, g[:, None, :], -g[:, None, :])
    out_ref[0] = jnp.sum(signed, axis=-1) + cb_ref[...]


def kernel(E, q_weight, c_weight, q_bias, cls_bias):
    del q_bias
    lg = pl.pallas_call(
        _body,
        grid=(B,),
        in_specs=[
            pl.BlockSpec((1, T, D), lambda b: (b, 0, 0)),
            pl.BlockSpec((K, D), lambda b: (0, 0)),
            pl.BlockSpec((K, G, D), lambda b: (0, 0, 0)),
            pl.BlockSpec((K, G), lambda b: (0, 0)),
        ],
        out_specs=pl.BlockSpec((1, K, G), lambda b: (b, 0, 0)),
        out_shape=jax.ShapeDtypeStruct((B, K, G), jnp.float32),
    )(E, q_weight, c_weight, cls_bias)
    return lg.reshape(B, K * G)[:, :NUM_CLASSES]
